# baseline (speedup 1.0000x reference)
import jax
import jax.numpy as jnp
from jax.experimental import pallas as pl
from jax.experimental.pallas import tpu as pltpu

N = 10000
F = 128
CHUNK = 1000
NCH = N // CHUNK


def _body(x_hbm, w1_ref, b1_ref, w2_ref, b2_ref, o_hbm, in_buf, out_buf, in_sem, out_sem):
    def get_in(slot, i):
        return pltpu.make_async_copy(
            x_hbm.at[pl.ds(i * CHUNK, CHUNK), :],
            in_buf.at[slot],
            in_sem.at[slot],
        )

    def put_out(slot, i):
        return pltpu.make_async_copy(
            out_buf.at[slot],
            o_hbm.at[pl.ds(i * CHUNK, CHUNK), :],
            out_sem.at[slot],
        )

    get_in(0, 0).start()

    def loop(i, carry):
        slot = jax.lax.rem(i, 2)

        @pl.when(i + 1 < NCH)
        def _():
            get_in(1 - slot, i + 1).start()

        get_in(slot, i).wait()
        h = jnp.dot(in_buf[slot], w1_ref[...], preferred_element_type=jnp.float32)
        h = jnp.maximum(h + b1_ref[...], 0.0)
        out = jnp.dot(h, w2_ref[...], preferred_element_type=jnp.float32)
        out = out + b2_ref[...]
        m = jnp.max(out, axis=-1, keepdims=True)
        l = out - m
        y = l - jnp.log(jnp.sum(jnp.exp(l), axis=-1, keepdims=True))

        @pl.when(i >= 2)
        def _():
            put_out(slot, i - 2).wait()

        out_buf[slot] = y
        put_out(slot, i).start()
        return carry

    jax.lax.fori_loop(0, NCH, loop, 0)
    put_out((NCH - 2) % 2, NCH - 2).wait()
    put_out((NCH - 1) % 2, NCH - 1).wait()


def kernel(x, W1, b1, W2, b2, edge_index):
    del edge_index
    w1t = W1.T
    w2t = W2.T
    b1r = b1.reshape(1, F)
    b2r = b2.reshape(1, F)
    return pl.pallas_call(
        _body,
        in_specs=[
            pl.BlockSpec(memory_space=pltpu.MemorySpace.HBM),
            pl.BlockSpec(memory_space=pltpu.MemorySpace.VMEM),
            pl.BlockSpec(memory_space=pltpu.MemorySpace.VMEM),
            pl.BlockSpec(memory_space=pltpu.MemorySpace.VMEM),
            pl.BlockSpec(memory_space=pltpu.MemorySpace.VMEM),
        ],
        out_specs=pl.BlockSpec(memory_space=pltpu.MemorySpace.HBM),
        out_shape=jax.ShapeDtypeStruct((N, F), jnp.float32),
        scratch_shapes=[
            pltpu.VMEM((2, CHUNK, F), jnp.float32),
            pltpu.VMEM((2, CHUNK, F), jnp.float32),
            pltpu.SemaphoreType.DMA((2,)),
            pltpu.SemaphoreType.DMA((2,)),
        ],
    )(x, w1t, b1r, w2t, b2r)


# speedup vs baseline: 1.1443x; 1.1443x over previous
import jax
import jax.numpy as jnp
from jax.experimental import pallas as pl
from jax.experimental.pallas import tpu as pltpu

N = 10000
F = 128
CHUNK = 1000
NCH = N // CHUNK
NSLOT = 4
PREFETCH = 3


def _body(x_hbm, w1_ref, b1_ref, w2_ref, b2_ref, o_hbm, in_buf, out_buf, in_sem, out_sem):
    def get_in(slot, i):
        return pltpu.make_async_copy(
            x_hbm.at[pl.ds(i * CHUNK, CHUNK), :],
            in_buf.at[slot],
            in_sem.at[slot],
        )

    def put_out(slot, i):
        return pltpu.make_async_copy(
            out_buf.at[slot],
            o_hbm.at[pl.ds(i * CHUNK, CHUNK), :],
            out_sem.at[slot],
        )

    for k in range(min(PREFETCH, NCH)):
        get_in(k % NSLOT, k).start()

    def loop(i, carry):
        slot = jax.lax.rem(i, NSLOT)

        @pl.when(i + PREFETCH < NCH)
        def _():
            get_in(jax.lax.rem(i + PREFETCH, NSLOT), i + PREFETCH).start()

        get_in(slot, i).wait()
        h = jnp.dot(in_buf[slot], w1_ref[...], preferred_element_type=jnp.float32)
        h = jnp.maximum(h + b1_ref[...], 0.0)
        out = jnp.dot(h, w2_ref[...], preferred_element_type=jnp.float32)
        out = out + b2_ref[...]
        m = jnp.max(out, axis=-1, keepdims=True)
        l = out - m
        y = l - jnp.log(jnp.sum(jnp.exp(l), axis=-1, keepdims=True))

        @pl.when(i >= NSLOT)
        def _():
            put_out(slot, i - NSLOT).wait()

        out_buf[slot] = y
        put_out(slot, i).start()
        return carry

    jax.lax.fori_loop(0, NCH, loop, 0)
    for k in range(max(0, NCH - NSLOT), NCH):
        put_out(k % NSLOT, k).wait()


def kernel(x, W1, b1, W2, b2, edge_index):
    del edge_index
    w1t = W1.T
    w2t = W2.T
    b1r = b1.reshape(1, F)
    b2r = b2.reshape(1, F)
    return pl.pallas_call(
        _body,
        in_specs=[
            pl.BlockSpec(memory_space=pltpu.MemorySpace.HBM),
            pl.BlockSpec(memory_space=pltpu.MemorySpace.VMEM),
            pl.BlockSpec(memory_space=pltpu.MemorySpace.VMEM),
            pl.BlockSpec(memory_space=pltpu.MemorySpace.VMEM),
            pl.BlockSpec(memory_space=pltpu.MemorySpace.VMEM),
        ],
        out_specs=pl.BlockSpec(memory_space=pltpu.MemorySpace.HBM),
        out_shape=jax.ShapeDtypeStruct((N, F), jnp.float32),
        scratch_shapes=[
            pltpu.VMEM((NSLOT, CHUNK, F), jnp.float32),
            pltpu.VMEM((NSLOT, CHUNK, F), jnp.float32),
            pltpu.SemaphoreType.DMA((NSLOT,)),
            pltpu.SemaphoreType.DMA((NSLOT,)),
        ],
    )(x, w1t, b1r, w2t, b2r)


# speedup vs baseline: 1.3549x; 1.1841x over previous
import jax
import jax.numpy as jnp
from jax.experimental import pallas as pl
from jax.experimental.pallas import tpu as pltpu

N = 10000
F = 128
CHUNK = 2000
NCH = N // CHUNK
NSLOT = 4
PREFETCH = 3


def _body(x_hbm, w1_ref, b1_ref, w2_ref, b2_ref, o_hbm, in_buf, out_buf, in_sem, out_sem):
    def get_in(slot, i):
        return pltpu.make_async_copy(
            x_hbm.at[pl.ds(i * CHUNK, CHUNK), :],
            in_buf.at[slot],
            in_sem.at[slot],
        )

    def put_out(slot, i):
        return pltpu.make_async_copy(
            out_buf.at[slot],
            o_hbm.at[pl.ds(i * CHUNK, CHUNK), :],
            out_sem.at[slot],
        )

    for k in range(min(PREFETCH, NCH)):
        get_in(k % NSLOT, k).start()

    def loop(i, carry):
        slot = jax.lax.rem(i, NSLOT)

        @pl.when(i + PREFETCH < NCH)
        def _():
            get_in(jax.lax.rem(i + PREFETCH, NSLOT), i + PREFETCH).start()

        get_in(slot, i).wait()
        h = jnp.dot(in_buf[slot], w1_ref[...], preferred_element_type=jnp.float32)
        h = jnp.maximum(h + b1_ref[...], 0.0)
        out = jnp.dot(h, w2_ref[...], preferred_element_type=jnp.float32)
        out = out + b2_ref[...]
        m = jnp.max(out, axis=-1, keepdims=True)
        l = out - m
        y = l - jnp.log(jnp.sum(jnp.exp(l), axis=-1, keepdims=True))

        @pl.when(i >= NSLOT)
        def _():
            put_out(slot, i - NSLOT).wait()

        out_buf[slot] = y
        put_out(slot, i).start()
        return carry

    jax.lax.fori_loop(0, NCH, loop, 0)
    for k in range(max(0, NCH - NSLOT), NCH):
        put_out(k % NSLOT, k).wait()


def kernel(x, W1, b1, W2, b2, edge_index):
    del edge_index
    w1t = W1.T
    w2t = W2.T
    b1r = b1.reshape(1, F)
    b2r = b2.reshape(1, F)
    return pl.pallas_call(
        _body,
        in_specs=[
            pl.BlockSpec(memory_space=pltpu.MemorySpace.HBM),
            pl.BlockSpec(memory_space=pltpu.MemorySpace.VMEM),
            pl.BlockSpec(memory_space=pltpu.MemorySpace.VMEM),
            pl.BlockSpec(memory_space=pltpu.MemorySpace.VMEM),
            pl.BlockSpec(memory_space=pltpu.MemorySpace.VMEM),
        ],
        out_specs=pl.BlockSpec(memory_space=pltpu.MemorySpace.HBM),
        out_shape=jax.ShapeDtypeStruct((N, F), jnp.float32),
        scratch_shapes=[
            pltpu.VMEM((NSLOT, CHUNK, F), jnp.float32),
            pltpu.VMEM((NSLOT, CHUNK, F), jnp.float32),
            pltpu.SemaphoreType.DMA((NSLOT,)),
            pltpu.SemaphoreType.DMA((NSLOT,)),
        ],
    )(x, w1t, b1r, w2t, b2r)


# speedup vs baseline: 1.5453x; 1.1405x over previous
import jax
import jax.numpy as jnp
from jax.experimental import pallas as pl
from jax.experimental.pallas import tpu as pltpu

N = 10000
F = 128
BLOCK = 5000


def _body(x_ref, w1_ref, b1_ref, w2_ref, b2_ref, o_ref):
    xb = x_ref[...].astype(jnp.bfloat16)
    h = jnp.dot(xb, w1_ref[...], preferred_element_type=jnp.float32)
    h = jnp.maximum(h + b1_ref[...], 0.0)
    out = jnp.dot(h.astype(jnp.bfloat16), w2_ref[...], preferred_element_type=jnp.float32)
    out = out + b2_ref[...]
    m = jnp.max(out, axis=-1, keepdims=True)
    l = out - m
    o_ref[...] = l - jnp.log(jnp.sum(jnp.exp(l), axis=-1, keepdims=True))


def kernel(x, W1, b1, W2, b2, edge_index):
    del edge_index
    w1t = W1.T.astype(jnp.bfloat16)
    w2t = W2.T.astype(jnp.bfloat16)
    b1r = b1.reshape(1, F)
    b2r = b2.reshape(1, F)
    grid = (N // BLOCK,)
    return pl.pallas_call(
        _body,
        grid=grid,
        in_specs=[
            pl.BlockSpec((BLOCK, F), lambda i: (i, 0)),
            pl.BlockSpec((F, F), lambda i: (0, 0)),
            pl.BlockSpec((1, F), lambda i: (0, 0)),
            pl.BlockSpec((F, F), lambda i: (0, 0)),
            pl.BlockSpec((1, F), lambda i: (0, 0)),
        ],
        out_specs=pl.BlockSpec((BLOCK, F), lambda i: (i, 0)),
        out_shape=jax.ShapeDtypeStruct((N, F), jnp.float32),
        compiler_params=pltpu.CompilerParams(
            dimension_semantics=("arbitrary",),
        ),
    )(x, w1t, b1r, w2t, b2r)
